# baseline (speedup 1.0000x reference)
import jax
import jax.numpy as jnp
from jax.experimental import pallas as pl

_N = 4096
_R = 256
_K = 10


def _bitonic_topk_plan(n, k):
    ex = []
    kk = 2
    while kk <= n:
        j = kk // 2
        while j >= 1:
            for i in range(n):
                l = i ^ j
                if l > i:
                    ex.append((i, l, (i & kk) == 0))
            j //= 2
        kk *= 2
    needed = set(range(k))
    plan = []
    for (i, l, d) in reversed(ex):
        ni, nl = i in needed, l in needed
        if not (ni or nl):
            continue
        plan.append((i, l, d, ni, nl))
        needed.add(i)
        needed.add(l)
    plan.reverse()
    return plan


_PLAN16 = _bitonic_topk_plan(16, _K)


def _lane_topk(chunks, plan):
    a = list(chunks)
    for (i, l, desc, ni, nl) in plan:
        x, y = a[i], a[l]
        hi = jnp.maximum(x, y) if (ni if desc else nl) else None
        lo = jnp.minimum(x, y) if (nl if desc else ni) else None
        if desc:
            if ni:
                a[i] = hi
            if nl:
                a[l] = lo
        else:
            if ni:
                a[i] = lo
            if nl:
                a[l] = hi
    return a


def _threshold(s):
    r, n = s.shape
    nchunks = n // 128
    if nchunks == 32:
        chunks = [s[:, g * 128:(g + 1) * 128] for g in range(nchunks)]
        a = _lane_topk(chunks[:16], _PLAN16)
        b = _lane_topk(chunks[16:], _PLAN16)
        cands = jnp.concatenate(
            [jnp.maximum(a[i], b[_K - 1 - i]) for i in range(_K)], axis=1)
    else:
        cands = s

    v = jnp.max(cands, axis=1, keepdims=True)
    for _ in range(_K - 1):
        v = jnp.max(jnp.where(cands < v, cands, -1.0), axis=1, keepdims=True)
    c10 = jnp.sum(jnp.where(cands >= v, 1.0, 0.0), axis=1, keepdims=True)

    def _tie_exact(_):
        t = jnp.max(cands, axis=1, keepdims=True)
        c = jnp.sum(jnp.where(cands >= t, 1.0, 0.0), axis=1, keepdims=True)
        for _ in range(_K - 1):
            m = jnp.max(jnp.where(cands < t, cands, -1.0), axis=1,
                        keepdims=True)
            cnt = jnp.sum(jnp.where(cands >= m, 1.0, 0.0), axis=1,
                          keepdims=True)
            upd = c < float(_K)
            t = jnp.where(upd, m, t)
            c = jnp.where(upd, cnt, c)
        return t

    return jax.lax.cond(jnp.any(c10 != float(_K)), _tie_exact,
                        lambda _: v, None)


def _fused(wr_ref, wc_ref, o_ref):
    i = pl.program_id(0)
    wr = wr_ref[...]
    wc = wc_ref[...]
    r, n = wr.shape
    s = jnp.maximum(jnp.maximum(wr, 0.0), jnp.maximum(wc, 0.0).T)
    col = jax.lax.broadcasted_iota(jnp.int32, (r, n), 1)
    row = jax.lax.broadcasted_iota(jnp.int32, (r, n), 0) + i * r
    s = jnp.where(col == row, s + 1.0, s)
    t = _threshold(s)
    masked = jnp.where(s >= t, s, 0.0)
    ssum = jnp.sum(masked, axis=1, keepdims=True)
    o_ref[...] = masked * (1.0 / (ssum + 1e-8))


def kernel(W):
    n = W.shape[0]
    g = n // _R
    row_spec = pl.BlockSpec((_R, n), lambda i: (i, 0))
    col_spec = pl.BlockSpec((n, _R), lambda i: (0, i))
    return pl.pallas_call(
        _fused,
        grid=(g,),
        in_specs=[row_spec, col_spec],
        out_specs=row_spec,
        out_shape=jax.ShapeDtypeStruct((n, n), jnp.float32),
    )(W, W)


# speedup vs baseline: 1.1146x; 1.1146x over previous
import jax
import jax.numpy as jnp
from jax.experimental import pallas as pl

_N = 4096
_R = 256
_K = 10


def _bitonic_topk_plan(n, k):
    ex = []
    kk = 2
    while kk <= n:
        j = kk // 2
        while j >= 1:
            for i in range(n):
                l = i ^ j
                if l > i:
                    ex.append((i, l, (i & kk) == 0))
            j //= 2
        kk *= 2
    needed = set(range(k))
    plan = []
    for (i, l, d) in reversed(ex):
        ni, nl = i in needed, l in needed
        if not (ni or nl):
            continue
        plan.append((i, l, d, ni, nl))
        needed.add(i)
        needed.add(l)
    plan.reverse()
    return plan


_PLAN16 = _bitonic_topk_plan(16, _K)


def _lane_topk(chunks, plan):
    a = list(chunks)
    for (i, l, desc, ni, nl) in plan:
        x, y = a[i], a[l]
        hi = jnp.maximum(x, y) if (ni if desc else nl) else None
        lo = jnp.minimum(x, y) if (nl if desc else ni) else None
        if desc:
            if ni:
                a[i] = hi
            if nl:
                a[l] = lo
        else:
            if ni:
                a[i] = lo
            if nl:
                a[l] = hi
    return a


def _threshold(s):
    r, n = s.shape
    nchunks = n // 128
    if nchunks == 32:
        chunks = [s[:, g * 128:(g + 1) * 128] for g in range(nchunks)]
        a = _lane_topk(chunks[:16], _PLAN16)
        b = _lane_topk(chunks[16:], _PLAN16)
        cands = jnp.concatenate(
            [jnp.maximum(a[i], b[_K - 1 - i]) for i in range(_K)], axis=1)
    else:
        cands = s

    t = jnp.max(cands, axis=1, keepdims=True)
    c = jnp.sum(jnp.where(cands >= t, 1.0, 0.0), axis=1, keepdims=True)
    for _ in range(_K - 1):
        m = jnp.max(jnp.where(cands < t, cands, -1.0), axis=1, keepdims=True)
        cnt = jnp.sum(jnp.where(cands >= m, 1.0, 0.0), axis=1, keepdims=True)
        upd = c < float(_K)
        t = jnp.where(upd, m, t)
        c = jnp.where(upd, cnt, c)
    return t


def _fused(wr_ref, wc_ref, o_ref):
    i = pl.program_id(0)
    wr = wr_ref[...]
    wc = wc_ref[...]
    r, n = wr.shape
    s = jnp.maximum(jnp.maximum(wr, 0.0), jnp.maximum(wc, 0.0).T)
    col = jax.lax.broadcasted_iota(jnp.int32, (r, n), 1)
    row = jax.lax.broadcasted_iota(jnp.int32, (r, n), 0) + i * r
    s = jnp.where(col == row, s + 1.0, s)
    t = _threshold(s)
    masked = jnp.where(s >= t, s, 0.0)
    ssum = jnp.sum(masked, axis=1, keepdims=True)
    o_ref[...] = masked * (1.0 / (ssum + 1e-8))


def kernel(W):
    n = W.shape[0]
    g = n // _R
    row_spec = pl.BlockSpec((_R, n), lambda i: (i, 0))
    col_spec = pl.BlockSpec((n, _R), lambda i: (0, i))
    return pl.pallas_call(
        _fused,
        grid=(g,),
        in_specs=[row_spec, col_spec],
        out_specs=row_spec,
        out_shape=jax.ShapeDtypeStruct((n, n), jnp.float32),
    )(W, W)
